# baseline (speedup 1.0000x reference)
import jax
import jax.numpy as jnp
from jax.experimental import pallas as pl
from jax.experimental.pallas import tpu as pltpu
from jax.experimental.pallas import tpu_sc as plsc

D = 1024
BLK_R = 8


def _sc_full(lvl, level_embeddings, band_embeddings, x2):
    mesh = plsc.VectorSubcoreMesh(core_axis_name="c", subcore_axis_name="s")

    @pl.kernel(
        out_type=jax.ShapeDtypeStruct(x2.shape, x2.dtype),
        mesh=mesh,
        scratch_types=[
            pltpu.VMEM((1, D), jnp.float32),
            pltpu.VMEM((1, D), jnp.float32),
            pltpu.VMEM((1, 1), jnp.int32),
        ],
    )
    def full_kernel(lvl_hbm, lev_hbm, band_hbm, x_hbm, o_hbm, bias_v, tmp_v, i_v):
        pltpu.sync_copy(lvl_hbm, i_v)
        pltpu.sync_copy(lev_hbm.at[i_v.at[0]], bias_v)
        pltpu.sync_copy(band_hbm.at[pl.ds(0, 1)], tmp_v)

        @pl.loop(0, D, step=16)
        def _(k):
            slc = (pl.ds(0, 1), pl.ds(k, 16))
            bias_v.at[*slc][...] = bias_v.at[*slc][...] + tmp_v.at[*slc][...]

        def body(in_v, out_v):
            @pl.loop(0, BLK_R)
            def _(r):
                @pl.loop(0, D, step=16)
                def _(c):
                    src = (pl.ds(r, 1), pl.ds(c, 16))
                    out_v.at[*src][...] = (
                        in_v.at[*src][...] + bias_v.at[pl.ds(0, 1), pl.ds(c, 16)][...]
                    )

        pltpu.emit_pipeline(
            body,
            grid=(x2.shape[0] // BLK_R,),
            in_specs=[pl.BlockSpec((BLK_R, D), index_map=lambda i: (i, 0))],
            out_specs=[pl.BlockSpec((BLK_R, D), index_map=lambda i: (i, 0))],
            core_axis_name=("c", "s"),
            dimension_semantics=(pltpu.PARALLEL,),
        )(x_hbm, o_hbm)

    return full_kernel(lvl, level_embeddings, band_embeddings, x2)


def kernel(x, level, level_embeddings, band_embeddings):
    b, s, d = x.shape
    rows = b * s
    x2 = x.reshape(rows, d)
    lvl = jnp.reshape(jnp.asarray(level, dtype=jnp.int32), (1, 1))
    out = _sc_full(lvl, level_embeddings, band_embeddings, x2)
    return out.reshape(b, s, d)


# speedup vs baseline: 4.8436x; 4.8436x over previous
import jax
import jax.numpy as jnp
from jax.experimental import pallas as pl
from jax.experimental.pallas import tpu as pltpu

BLOCK_ROWS = 3456


def _add_embed_kernel(lvl_ref, x_ref, lev_ref, band_ref, o_ref):
    lvl = lvl_ref[0]
    bias = lev_ref[pl.ds(lvl, 1), :] + band_ref[pl.ds(0, 1), :]
    o_ref[...] = x_ref[...] + bias


def kernel(x, level, level_embeddings, band_embeddings):
    b, s, d = x.shape
    rows = b * s
    x2 = x.reshape(rows, d)
    lvl = jnp.atleast_1d(jnp.asarray(level, dtype=jnp.int32))
    grid = (pl.cdiv(rows, BLOCK_ROWS),)
    out = pl.pallas_call(
        _add_embed_kernel,
        grid_spec=pltpu.PrefetchScalarGridSpec(
            num_scalar_prefetch=1,
            grid=grid,
            in_specs=[
                pl.BlockSpec((BLOCK_ROWS, d), lambda i, lvl: (i, 0)),
                pl.BlockSpec(level_embeddings.shape, lambda i, lvl: (0, 0)),
                pl.BlockSpec(band_embeddings.shape, lambda i, lvl: (0, 0)),
            ],
            out_specs=pl.BlockSpec((BLOCK_ROWS, d), lambda i, lvl: (i, 0)),
        ),
        out_shape=jax.ShapeDtypeStruct((rows, d), x.dtype),
        compiler_params=pltpu.CompilerParams(
            dimension_semantics=("parallel",),
        ),
    )(lvl, x2, level_embeddings, band_embeddings)
    return out.reshape(b, s, d)


# speedup vs baseline: 4.8455x; 1.0004x over previous
import jax
import jax.numpy as jnp
from jax.experimental import pallas as pl
from jax.experimental.pallas import tpu as pltpu

BLOCK_ROWS = 3744


def _add_embed_kernel(lvl_ref, x_ref, lev_ref, band_ref, o_ref):
    lvl = lvl_ref[0]
    bias = lev_ref[pl.ds(lvl, 1), :] + band_ref[pl.ds(0, 1), :]
    o_ref[...] = x_ref[...] + bias


def kernel(x, level, level_embeddings, band_embeddings):
    b, s, d = x.shape
    rows = b * s
    x2 = x.reshape(rows, d)
    lvl = jnp.atleast_1d(jnp.asarray(level, dtype=jnp.int32))
    grid = (pl.cdiv(rows, BLOCK_ROWS),)
    out = pl.pallas_call(
        _add_embed_kernel,
        grid_spec=pltpu.PrefetchScalarGridSpec(
            num_scalar_prefetch=1,
            grid=grid,
            in_specs=[
                pl.BlockSpec((BLOCK_ROWS, d), lambda i, lvl: (i, 0)),
                pl.BlockSpec(level_embeddings.shape, lambda i, lvl: (0, 0)),
                pl.BlockSpec(band_embeddings.shape, lambda i, lvl: (0, 0)),
            ],
            out_specs=pl.BlockSpec((BLOCK_ROWS, d), lambda i, lvl: (i, 0)),
        ),
        out_shape=jax.ShapeDtypeStruct((rows, d), x.dtype),
        compiler_params=pltpu.CompilerParams(
            dimension_semantics=("parallel",),
        ),
    )(lvl, x2, level_embeddings, band_embeddings)
    return out.reshape(b, s, d)
